# baseline (device time: 118052 ns/iter reference)
import jax
import jax.numpy as jnp
from jax import lax
from jax.experimental import pallas as pl
from jax.experimental.pallas import tpu as pltpu

N_DEV = 8
SUB = 2

RING = (0, 4, 7, 3, 2, 6, 5, 1)
POS = (0, 7, 4, 3, 1, 6, 5, 2)


def kernel(x, w_mat, scale_x, scale_w):
    m_per, k = x.shape
    _, n_per = w_mat.shape
    half = m_per // 2
    qrt = half // SUB

    my = lax.axis_index("i")
    ring = jnp.array(RING, jnp.int32)
    pos = jnp.array(POS, jnp.int32)[my]
    steps = jnp.arange(N_DEV - 1)
    rl = jnp.stack([ring[(pos + 1) % N_DEV], ring[(pos - 1) % N_DEV],
                    my.astype(jnp.int32)])
    cw_origins = ring[(pos - 1 - steps) % N_DEV]
    ccw_origins = ring[(pos + 1 + steps) % N_DEV]

    def body(x_ref, w_ref, sx_ref, sw_ref, rl_ref, cwo_ref, ccwo_ref,
             out_ref, cw_ref, ccw_ref, w_bf_ref,
             cw_send, cw_recv, ccw_send, ccw_recv):
        right = rl_ref[0]
        left = rl_ref[1]

        barrier_sem = pltpu.get_barrier_semaphore()
        pl.semaphore_signal(barrier_sem, inc=1, device_id=(left,),
                            device_id_type=pl.DeviceIdType.MESH)
        pl.semaphore_signal(barrier_sem, inc=1, device_id=(right,),
                            device_id_type=pl.DeviceIdType.MESH)
        pl.semaphore_wait(barrier_sem, 2)

        def mk(buf, row0, s, j, dst, send_sems, recv_sems):
            if s == 0:
                src = x_ref.at[pl.ds(row0 + j * qrt, qrt), :]
            else:
                src = buf.at[s, j]
            return pltpu.make_async_remote_copy(
                src_ref=src,
                dst_ref=buf.at[s + 1, j],
                send_sem=send_sems.at[s, j],
                recv_sem=recv_sems.at[s, j],
                device_id=(dst,),
                device_id_type=pl.DeviceIdType.MESH,
            )

        cw_rdmas = [[mk(cw_ref, 0, s, j, right, cw_send, cw_recv)
                     for j in range(SUB)] for s in range(N_DEV - 1)]
        ccw_rdmas = [[mk(ccw_ref, half, s, j, left, ccw_send, ccw_recv)
                      for j in range(SUB)] for s in range(N_DEV - 1)]

        for j in range(SUB):
            cw_rdmas[0][j].start()
            ccw_rdmas[0][j].start()

        w_bf_ref[:, :] = w_ref[:, :].astype(jnp.bfloat16)
        scale = sx_ref[0] * sw_ref[0]

        def gemm_store(row_start, chunk_i8):
            acc = lax.dot_general(
                chunk_i8.astype(jnp.bfloat16), w_bf_ref[:, :],
                (((1,), (0,)), ((), ())),
                preferred_element_type=jnp.float32,
            )
            y = acc * scale
            out_ref[pl.ds(row_start, chunk_i8.shape[0]), :] = (
                y * jax.nn.sigmoid(y))

        gemm_store(rl_ref[2] * m_per, x_ref[:, :])

        for s in range(N_DEV - 1):
            for j in range(SUB):
                cw_rdmas[s][j].wait_recv()
                if s < N_DEV - 2:
                    cw_rdmas[s + 1][j].start()
                ccw_rdmas[s][j].wait_recv()
                if s < N_DEV - 2:
                    ccw_rdmas[s + 1][j].start()
            gemm_store(cwo_ref[s] * m_per,
                       cw_ref[s + 1].reshape(half, k))
            gemm_store(ccwo_ref[s] * m_per + half,
                       ccw_ref[s + 1].reshape(half, k))

        for s in range(N_DEV - 1):
            for j in range(SUB):
                cw_rdmas[s][j].wait_send()
                ccw_rdmas[s][j].wait_send()

    return pl.pallas_call(
        body,
        out_shape=jax.ShapeDtypeStruct((N_DEV * m_per, n_per), jnp.float32),
        in_specs=[
            pl.BlockSpec(memory_space=pltpu.VMEM),
            pl.BlockSpec(memory_space=pltpu.VMEM),
            pl.BlockSpec(memory_space=pltpu.SMEM),
            pl.BlockSpec(memory_space=pltpu.SMEM),
            pl.BlockSpec(memory_space=pltpu.SMEM),
            pl.BlockSpec(memory_space=pltpu.SMEM),
            pl.BlockSpec(memory_space=pltpu.SMEM),
        ],
        out_specs=pl.BlockSpec(memory_space=pltpu.VMEM),
        scratch_shapes=[
            pltpu.VMEM((N_DEV, SUB, qrt, k), jnp.int8),
            pltpu.VMEM((N_DEV, SUB, qrt, k), jnp.int8),
            pltpu.VMEM((k, n_per), jnp.bfloat16),
            pltpu.SemaphoreType.DMA((N_DEV - 1, SUB)),
            pltpu.SemaphoreType.DMA((N_DEV - 1, SUB)),
            pltpu.SemaphoreType.DMA((N_DEV - 1, SUB)),
            pltpu.SemaphoreType.DMA((N_DEV - 1, SUB)),
        ],
        compiler_params=pltpu.CompilerParams(
            collective_id=0,
            vmem_limit_bytes=60 * 1024 * 1024,
        ),
    )(x, w_mat, scale_x, scale_w, rl, cw_origins, ccw_origins)
